# baseline (speedup 1.0000x reference)
import functools

import numpy as np
import jax
import jax.numpy as jnp
from jax import lax
from jax.experimental import pallas as pl
from jax.experimental.pallas import tpu as pltpu
from jax.experimental.pallas import tpu_sc as plsc

_MAX_LEN = 200
_EMB_DIM = 64
_CHUNK = 100
_NW = 32


def _make_pe_np():
    pos = np.expand_dims(np.arange(_MAX_LEN), 1)
    pe = pos / np.power(
        1000, 2 * np.expand_dims(np.arange(_EMB_DIM) // 2, 0) / _EMB_DIM
    )
    pe = pe.astype(np.float64)
    pe[:, 0::2] = np.sin(pe[:, 0::2])
    pe[:, 1::2] = np.cos(pe[:, 1::2])
    return pe.astype(np.float32)


_PE = _make_pe_np()


def _emb_sc(table, xi, pe):
    n_chunks = xi.shape[0]
    per_w = n_chunks // _NW
    n_blk = 4
    blk = per_w // n_blk
    mesh = plsc.VectorSubcoreMesh(core_axis_name="c", subcore_axis_name="s")

    @functools.partial(
        pl.kernel,
        mesh=mesh,
        out_type=jax.ShapeDtypeStruct((n_chunks, _CHUNK, _EMB_DIM), jnp.float32),
        scratch_types=[
            pltpu.VMEM((blk, _CHUNK), jnp.int32),
            pltpu.VMEM((_MAX_LEN, _EMB_DIM), jnp.float32),
            pltpu.VMEM((2, _CHUNK, _EMB_DIM), jnp.float32),
            pltpu.SemaphoreType.DMA,
        ],
        compiler_params=pltpu.CompilerParams(use_tc_tiling_on_sc=False),
    )
    def k(table_h, xi_h, pe_h, out_h, idx_v, pe_v, rows_v, sem):
        cid = lax.axis_index("c")
        sid = lax.axis_index("s")
        wid = sid * 2 + cid
        base = wid * per_w
        pltpu.sync_copy(pe_h, pe_v)

        def blk_body(b_i, carry):
            c0 = base + b_i * blk
            pltpu.sync_copy(xi_h.at[pl.ds(c0, blk)], idx_v)

            def step(g, carry2):
                for b in range(2):
                    cc = g * 2 + b
                    pltpu.async_copy(
                        table_h.at[idx_v.at[cc]], rows_v.at[b], sem
                    ).wait()

                    def row(i, carry3):
                        for j in range(4):
                            sl = pl.ds(j * 16, 16)
                            rows_v[b, i, sl] = (
                                rows_v[b, i, sl] + pe_v[b * _CHUNK + i, sl]
                            )
                        return carry3

                    lax.fori_loop(0, _CHUNK, row, 0)
                    pltpu.sync_copy(rows_v.at[b], out_h.at[c0 + cc])
                return carry2

            lax.fori_loop(0, blk // 2, step, 0)
            return carry

        lax.fori_loop(0, n_blk, blk_body, 0)

    return k(table, xi, pe)


def kernel(x, table):
    b, t = x.shape
    xi = x.astype(jnp.int32).reshape(-1, _CHUNK)
    pe = jnp.asarray(_PE)
    out = _emb_sc(table, xi, pe)
    return out.reshape(b, t, _EMB_DIM)


# speedup vs baseline: 1.3885x; 1.3885x over previous
import functools

import numpy as np
import jax
import jax.numpy as jnp
from jax import lax
from jax.experimental import pallas as pl
from jax.experimental.pallas import tpu as pltpu
from jax.experimental.pallas import tpu_sc as plsc

_MAX_LEN = 200
_EMB_DIM = 64
_CHUNK = 100
_NW = 32
_NBUF = 8
_LAG = 4
_BLK = 128


def _make_pe_np():
    pos = np.expand_dims(np.arange(_MAX_LEN), 1)
    pe = pos / np.power(
        1000, 2 * np.expand_dims(np.arange(_EMB_DIM) // 2, 0) / _EMB_DIM
    )
    pe = pe.astype(np.float64)
    pe[:, 0::2] = np.sin(pe[:, 0::2])
    pe[:, 1::2] = np.cos(pe[:, 1::2])
    return pe.astype(np.float32)


_PE = _make_pe_np()


def _emb_sc(table, xi, pe):
    n_chunks = xi.shape[0]
    per_w = n_chunks // _NW
    n_blks = per_w // _BLK
    rpb = _BLK // _NBUF
    n_rounds = per_w // _NBUF
    mesh = plsc.VectorSubcoreMesh(core_axis_name="c", subcore_axis_name="s")

    @functools.partial(
        pl.kernel,
        mesh=mesh,
        out_type=jax.ShapeDtypeStruct((n_chunks, _CHUNK, _EMB_DIM), jnp.float32),
        scratch_types=[
            pltpu.VMEM((2, _BLK, _CHUNK), jnp.int32),
            pltpu.VMEM((_MAX_LEN, _EMB_DIM), jnp.float32),
            pltpu.VMEM((_NBUF, _CHUNK, _EMB_DIM), jnp.float32),
            pltpu.SemaphoreType.DMA,
            pltpu.SemaphoreType.DMA((_NBUF,)),
            pltpu.SemaphoreType.DMA((_NBUF,)),
        ],
        compiler_params=pltpu.CompilerParams(use_tc_tiling_on_sc=False),
    )
    def k(table_h, xi_h, pe_h, out_h, idx_v, pe_v, rows_v, sem_ix, sem_g, sem_o):
        cid = lax.axis_index("c")
        sid = lax.axis_index("s")
        wid = sid * 2 + cid
        base = wid * per_w
        pltpu.sync_copy(pe_h, pe_v)

        def start_gather(b, buf, ccm):
            pltpu.async_copy(table_h.at[idx_v.at[buf, ccm]], rows_v.at[b], sem_g.at[b])

        def wait_gather(b):
            pltpu.make_async_copy(
                table_h.at[idx_v.at[0, 0]], rows_v.at[b], sem_g.at[b]
            ).wait()

        def start_store(b, g_cd):
            pltpu.async_copy(rows_v.at[b], out_h.at[g_cd], sem_o.at[b])

        def wait_store(b):
            pltpu.make_async_copy(rows_v.at[b], out_h.at[0], sem_o.at[b]).wait()

        def add_pe(b, parity):
            def row(i, carry):
                for u in range(2):
                    for j in range(4):
                        sl = pl.ds(j * 16, 16)
                        rows_v[b, 2 * i + u, sl] = (
                            rows_v[b, 2 * i + u, sl]
                            + pe_v[parity * _CHUNK + 2 * i + u, sl]
                        )
                return carry

            lax.fori_loop(0, _CHUNK // 2, row, 0)

        pltpu.sync_copy(xi_h.at[pl.ds(base, _BLK)], idx_v.at[0])
        pltpu.async_copy(xi_h.at[pl.ds(base + _BLK, _BLK)], idx_v.at[1], sem_ix)

        for b in range(_NBUF):
            start_gather(b, 0, b)
            if b >= _LAG:
                bb = b - _LAG
                wait_gather(bb)
                add_pe(bb, b % 2)
                start_store(bb, base + b - _LAG)

        def round_body(r, carry):
            k_blk = r // rpb
            buf = lax.rem(k_blk, 2)

            @pl.when(lax.rem(r, rpb) == 0)
            def _():
                pltpu.make_async_copy(
                    xi_h.at[pl.ds(base, _BLK)], idx_v.at[buf], sem_ix
                ).wait()

            for b in range(_NBUF):
                cc = r * _NBUF + b
                ccm = lax.rem(cc, _BLK)
                wait_store(b)
                start_gather(b, buf, ccm)
                bb = (b + _LAG) % _NBUF
                cd = cc - _LAG
                wait_gather(bb)
                add_pe(bb, b % 2)
                start_store(bb, base + cd)
                if b == _LAG - 1:
                    @pl.when((lax.rem(r, rpb) == 0) & (k_blk < n_blks - 1))
                    def _():
                        pltpu.async_copy(
                            xi_h.at[pl.ds(base + (k_blk + 1) * _BLK, _BLK)],
                            idx_v.at[1 - buf],
                            sem_ix,
                        )
            return carry

        lax.fori_loop(1, n_rounds, round_body, 0)

        for bb in range(_NBUF - _LAG, _NBUF):
            cd = per_w - _NBUF + bb
            wait_gather(bb)
            add_pe(bb, bb % 2)
            start_store(bb, base + cd)
        for b in range(_NBUF):
            wait_store(b)

    return k(table, xi, pe)


def kernel(x, table):
    b, t = x.shape
    xi = x.astype(jnp.int32).reshape(-1, _CHUNK)
    pe = jnp.asarray(_PE)
    out = _emb_sc(table, xi, pe)
    return out.reshape(b, t, _EMB_DIM)


# speedup vs baseline: 1.4682x; 1.0574x over previous
import functools

import numpy as np
import jax
import jax.numpy as jnp
from jax import lax
from jax.experimental import pallas as pl
from jax.experimental.pallas import tpu as pltpu
from jax.experimental.pallas import tpu_sc as plsc

_MAX_LEN = 200
_EMB_DIM = 64
_NW = 32
_NBUF = 4
_LAG = 2
_BLKR = 64


def _make_pe_np():
    pos = np.expand_dims(np.arange(_MAX_LEN), 1)
    pe = pos / np.power(
        1000, 2 * np.expand_dims(np.arange(_EMB_DIM) // 2, 0) / _EMB_DIM
    )
    pe = pe.astype(np.float64)
    pe[:, 0::2] = np.sin(pe[:, 0::2])
    pe[:, 1::2] = np.cos(pe[:, 1::2])
    return pe.astype(np.float32)


_PE = _make_pe_np()


def _emb_sc(table, xi, pe):
    n_rows = xi.shape[0]
    rows_per_w = n_rows // _NW
    n_blks = rows_per_w // _BLKR
    rpb = _BLKR // _NBUF
    n_rounds = rows_per_w // _NBUF
    mesh = plsc.VectorSubcoreMesh(core_axis_name="c", subcore_axis_name="s")

    @functools.partial(
        pl.kernel,
        mesh=mesh,
        out_type=jax.ShapeDtypeStruct((n_rows, _MAX_LEN, _EMB_DIM), jnp.float32),
        scratch_types=[
            pltpu.VMEM((2, _BLKR, _MAX_LEN), jnp.int32),
            pltpu.VMEM((_MAX_LEN, _EMB_DIM), jnp.float32),
            pltpu.VMEM((_NBUF, _MAX_LEN, _EMB_DIM), jnp.float32),
            pltpu.SemaphoreType.DMA,
            pltpu.SemaphoreType.DMA((_NBUF,)),
            pltpu.SemaphoreType.DMA((_NBUF,)),
        ],
        compiler_params=pltpu.CompilerParams(use_tc_tiling_on_sc=False),
    )
    def k(table_h, xi_h, pe_h, out_h, idx_v, pe_v, rows_v, sem_ix, sem_g, sem_o):
        cid = lax.axis_index("c")
        sid = lax.axis_index("s")
        wid = sid * 2 + cid
        base_row = wid * rows_per_w
        pltpu.sync_copy(pe_h, pe_v)

        def start_gather(b, buf, rr):
            pltpu.async_copy(
                table_h.at[idx_v.at[buf, rr]], rows_v.at[b], sem_g.at[b]
            )

        def wait_gather(b):
            pltpu.make_async_copy(
                table_h.at[idx_v.at[0, 0]], rows_v.at[b], sem_g.at[b]
            ).wait()

        def start_store(b, row):
            pltpu.async_copy(rows_v.at[b], out_h.at[row], sem_o.at[b])

        def wait_store(b):
            pltpu.make_async_copy(rows_v.at[b], out_h.at[0], sem_o.at[b]).wait()

        def add_pe(b):
            def row_it(i, carry):
                for u in range(2):
                    for j in range(4):
                        sl = pl.ds(j * 16, 16)
                        rows_v[b, 2 * i + u, sl] = (
                            rows_v[b, 2 * i + u, sl] + pe_v[2 * i + u, sl]
                        )
                return carry

            lax.fori_loop(0, _MAX_LEN // 2, row_it, 0)

        pltpu.sync_copy(xi_h.at[pl.ds(base_row, _BLKR)], idx_v.at[0])
        pltpu.async_copy(
            xi_h.at[pl.ds(base_row + _BLKR, _BLKR)], idx_v.at[1], sem_ix
        )

        for b in range(_NBUF):
            start_gather(b, 0, b)
            if b >= _LAG:
                bb = b - _LAG
                wait_gather(bb)
                add_pe(bb)
                start_store(bb, base_row + b - _LAG)

        def round_body(r, carry):
            k_blk = r // rpb
            buf = lax.rem(k_blk, 2)

            @pl.when(lax.rem(r, rpb) == 0)
            def _():
                pltpu.make_async_copy(
                    xi_h.at[pl.ds(base_row, _BLKR)], idx_v.at[buf], sem_ix
                ).wait()

            for b in range(_NBUF):
                cc = r * _NBUF + b
                rr = lax.rem(cc, _BLKR)
                wait_store(b)
                start_gather(b, buf, rr)
                bb = (b + _LAG) % _NBUF
                cd = cc - _LAG
                wait_gather(bb)
                add_pe(bb)
                start_store(bb, base_row + cd)
                if b == _LAG - 1:
                    @pl.when((lax.rem(r, rpb) == 0) & (k_blk < n_blks - 1))
                    def _():
                        pltpu.async_copy(
                            xi_h.at[pl.ds(base_row + (k_blk + 1) * _BLKR, _BLKR)],
                            idx_v.at[1 - buf],
                            sem_ix,
                        )
            return carry

        lax.fori_loop(1, n_rounds, round_body, 0)

        for bb in range(_NBUF - _LAG, _NBUF):
            cd = rows_per_w - _NBUF + bb
            wait_gather(bb)
            add_pe(bb)
            start_store(bb, base_row + cd)
        for b in range(_NBUF):
            wait_store(b)

    return k(table, xi, pe)


def kernel(x, table):
    xi = x.astype(jnp.int32)
    pe = jnp.asarray(_PE)
    return _emb_sc(table, xi, pe)
